# baseline (speedup 1.0000x reference)
import functools

import jax
import jax.numpy as jnp
from jax import lax
from jax.experimental import pallas as pl
from jax.experimental.pallas import tpu as pltpu
from jax.experimental.pallas import tpu_sc as plsc

K_NB = 64
N_KEYS = 100000
N_PAD = 100352
NB = 784
NB_REAL = 782
Q = 1024
D = 128
W = 1024
SB_PER_W = W // 128
CAP = 128
NW = 32
QPW = Q // NW
PAD_VAL = -1e30


def _a_body(q_ref, kt_ref, d_ref, mx_ref):
    q = q_ref[...]
    qsq = jnp.sum(q * q, axis=1, keepdims=True)
    ms = []
    for c in range(SB_PER_W):
        kt = kt_ref[:, c * 128:(c + 1) * 128]
        ksq = jnp.sum(kt * kt, axis=0, keepdims=True)
        dots = jnp.dot(q, kt, preferred_element_type=jnp.float32)
        s = -(qsq + ksq - 2.0 * dots)
        d_ref[c] = s
        ms.append(jnp.max(s, axis=1, keepdims=True))
    mx_ref[0] = jnp.concatenate(ms, axis=1)


def _scores(queries, kt):
    grid = N_PAD // W
    return pl.pallas_call(
        _a_body,
        grid=(grid,),
        in_specs=[
            pl.BlockSpec((Q, D), lambda j: (0, 0)),
            pl.BlockSpec((D, W), lambda j: (0, j)),
        ],
        out_specs=[
            pl.BlockSpec((SB_PER_W, Q, 128), lambda j: (j, 0, 0)),
            pl.BlockSpec((1, Q, SB_PER_W), lambda j: (j, 0, 0)),
        ],
        out_shape=[
            jax.ShapeDtypeStruct((NB, Q, 128), jnp.float32),
            jax.ShapeDtypeStruct((N_PAD // W, Q, SB_PER_W), jnp.float32),
        ],
    )(queries, kt)


def _b_body(mx_ref, t_ref):
    mx = mx_ref[...]
    lo = jnp.min(mx, axis=1, keepdims=True)
    hi = jnp.max(mx, axis=1, keepdims=True) + 1.0

    def it(_, lh):
        lo, hi = lh
        mid = 0.5 * (lo + hi)
        cnt = jnp.sum((mx >= mid).astype(jnp.float32), axis=1, keepdims=True)
        pred = cnt >= float(K_NB)
        return (jnp.where(pred, mid, lo), jnp.where(pred, hi, mid))

    lo, hi = lax.fori_loop(0, 26, it, (lo, hi))
    t_ref[...] = lo


def _threshold(mx):
    return pl.pallas_call(
        _b_body,
        in_specs=[pl.BlockSpec((Q, NB_REAL), lambda: (0, 0))],
        out_specs=pl.BlockSpec((Q, 1), lambda: (0, 0)),
        out_shape=jax.ShapeDtypeStruct((Q, 1), jnp.float32),
    )(mx)


def _sc_body(d2_hbm, mx_hbm, t_hbm, ov_hbm, oi_hbm,
             mx_v, t_v, ids_v, rows_v, val_v, idx_v, sem):
    wid = lax.axis_index("s") * 2 + lax.axis_index("c")
    qbase = wid * QPW
    pltpu.sync_copy(t_hbm, t_v)

    def qbody(qi, _):
        q = qbase + qi
        pltpu.sync_copy(mx_hbm.at[q], mx_v)
        tq = plsc.load_gather(t_v, (jnp.full((16,), q, jnp.int32),))

        for jj in range(CAP // 16):
            off = jj * 16
            ids_v[pl.ds(off, 16)] = (off + lax.iota(jnp.int32, 16)) * Q + q
            val_v[pl.ds(off, 16)] = jnp.full((16,), PAD_VAL, jnp.float32)
            idx_v[pl.ds(off, 16)] = jnp.full((16,), 2 ** 30, jnp.int32)

        def mxbody(j, cur):
            m = mx_v[pl.ds(j * 16, 16)]
            mask = m >= tq
            cnt = plsc.cumsum(jnp.where(mask, 1, 0).astype(jnp.int32))
            pos = cur + cnt - 1
            mask = mask & (pos < CAP)
            rid = (j * 16 + lax.iota(jnp.int32, 16)) * Q + q
            plsc.store_scatter(ids_v, (pos,), rid, mask=mask)
            return cur + plsc.all_reduce_population_count(mask)

        cur = lax.fori_loop(0, NB // 16, mxbody, jnp.zeros((16,), jnp.int32))
        n_sb = jnp.minimum(jnp.max(cur), CAP)

        pltpu.async_copy(d2_hbm.at[ids_v], rows_v, sem).wait()

        def rbody(r, cur):
            rid = plsc.load_gather(ids_v, (jnp.full((16,), r, jnp.int32),))
            kbase = ((rid - q) >> 10) * 128
            for c in range(8):
                v = rows_v[r, pl.ds(c * 16, 16)]
                mask = v >= tq

                def _slow(cur, v=v, mask=mask, c=c, kbase=kbase):
                    cnt = plsc.cumsum(jnp.where(mask, 1, 0).astype(jnp.int32))
                    pos = cur + cnt - 1
                    m2 = mask & (pos < CAP)
                    plsc.store_scatter(val_v, (pos,), v, mask=m2)
                    kidx = kbase + c * 16 + lax.iota(jnp.int32, 16)
                    plsc.store_scatter(idx_v, (pos,), kidx, mask=m2)
                    return cur + plsc.all_reduce_population_count(m2)

                cur = lax.cond(jnp.any(mask), _slow, lambda cur: cur, cur)
            return cur

        lax.fori_loop(0, n_sb, rbody, jnp.zeros((16,), jnp.int32))
        pltpu.sync_copy(val_v, ov_hbm.at[q])
        pltpu.sync_copy(idx_v, oi_hbm.at[q])
        return 0

    lax.fori_loop(0, QPW, qbody, 0)


def _sc_filter(d2, mx, t):
    mesh = plsc.VectorSubcoreMesh(core_axis_name="c", subcore_axis_name="s")
    f = pl.kernel(
        _sc_body,
        out_type=[
            jax.ShapeDtypeStruct((Q, CAP), jnp.float32),
            jax.ShapeDtypeStruct((Q, CAP), jnp.int32),
        ],
        mesh=mesh,
        compiler_params=pltpu.CompilerParams(needs_layout_passes=False),
        scratch_types=[
            pltpu.VMEM((NB,), jnp.float32),
            pltpu.VMEM((Q,), jnp.float32),
            pltpu.VMEM((CAP,), jnp.int32),
            pltpu.VMEM((CAP, 128), jnp.float32),
            pltpu.VMEM((CAP,), jnp.float32),
            pltpu.VMEM((CAP,), jnp.int32),
            pltpu.SemaphoreType.DMA,
        ],
    )
    return f(d2, mx, t)


def _d_body(v_ref, i_ref, ov_ref, oi_ref):
    v = v_ref[...]
    ci = i_ref[...]
    vs, ix = [], []
    for _ in range(K_NB):
        m = jnp.max(v, axis=1, keepdims=True)
        eq = v == m
        cand = jnp.where(eq, ci, 2 ** 30)
        si = jnp.min(cand, axis=1, keepdims=True)
        vs.append(m)
        ix.append(si)
        v = jnp.where(eq & (ci == si), PAD_VAL, v)
    ov_ref[...] = jnp.concatenate(vs, axis=1)
    oi_ref[...] = jnp.concatenate(ix, axis=1)


def _final_topk(cv, cidx):
    return pl.pallas_call(
        _d_body,
        in_specs=[
            pl.BlockSpec((Q, CAP), lambda: (0, 0)),
            pl.BlockSpec((Q, CAP), lambda: (0, 0)),
        ],
        out_specs=[
            pl.BlockSpec((Q, K_NB), lambda: (0, 0)),
            pl.BlockSpec((Q, K_NB), lambda: (0, 0)),
        ],
        out_shape=[
            jax.ShapeDtypeStruct((Q, K_NB), jnp.float32),
            jax.ShapeDtypeStruct((Q, K_NB), jnp.int32),
        ],
    )(cv, cidx)


def kernel(queries, keys):
    kp = jnp.pad(keys, ((0, N_PAD - N_KEYS), (0, 0)), constant_values=1e4)
    kt = kp.T
    d3, mx3 = _scores(queries, kt)
    mx = mx3.transpose(1, 0, 2).reshape(Q, NB)
    t = _threshold(mx[:, :NB_REAL])
    d2 = d3.reshape(NB * Q, 128)
    cv, cidx = _sc_filter(d2, mx, t.reshape(Q))
    return _final_topk(cv, cidx)


# speedup vs baseline: 1.3634x; 1.3634x over previous
import functools

import jax
import jax.numpy as jnp
from jax import lax
from jax.experimental import pallas as pl
from jax.experimental.pallas import tpu as pltpu
from jax.experimental.pallas import tpu_sc as plsc

K_NB = 64
N_KEYS = 100000
N_PAD = 100352
NB = 784
NB_REAL = 782
Q = 1024
D = 128
W = 1024
SB_PER_W = W // 128
CAP = 128
NW = 32
QPW = Q // NW
PAD_VAL = -1e30


def _a_body(q_ref, kt_ref, d_ref, mx_ref):
    q = q_ref[...]
    qsq = jnp.sum(q * q, axis=1, keepdims=True)
    ms = []
    for c in range(SB_PER_W):
        kt = kt_ref[:, c * 128:(c + 1) * 128]
        ksq = jnp.sum(kt * kt, axis=0, keepdims=True)
        dots = jnp.dot(q, kt, preferred_element_type=jnp.float32)
        s = -(qsq + ksq - 2.0 * dots)
        d_ref[c] = s
        ms.append(jnp.max(s, axis=1, keepdims=True))
    mx_ref[0] = jnp.concatenate(ms, axis=1)


def _scores(queries, kt):
    grid = N_PAD // W
    return pl.pallas_call(
        _a_body,
        grid=(grid,),
        in_specs=[
            pl.BlockSpec((Q, D), lambda j: (0, 0)),
            pl.BlockSpec((D, W), lambda j: (0, j)),
        ],
        out_specs=[
            pl.BlockSpec((SB_PER_W, Q, 128), lambda j: (j, 0, 0)),
            pl.BlockSpec((1, Q, SB_PER_W), lambda j: (j, 0, 0)),
        ],
        out_shape=[
            jax.ShapeDtypeStruct((NB, Q, 128), jnp.float32),
            jax.ShapeDtypeStruct((N_PAD // W, Q, SB_PER_W), jnp.float32),
        ],
    )(queries, kt)


def _b_body(mx_ref, t_ref):
    mx = mx_ref[...]
    lo = jnp.min(mx, axis=1, keepdims=True)
    hi = jnp.max(mx, axis=1, keepdims=True) + 1.0

    def it(_, lh):
        lo, hi = lh
        mid = 0.5 * (lo + hi)
        cnt = jnp.sum((mx >= mid).astype(jnp.float32), axis=1, keepdims=True)
        pred = cnt >= float(K_NB)
        return (jnp.where(pred, mid, lo), jnp.where(pred, hi, mid))

    lo, hi = lax.fori_loop(0, 26, it, (lo, hi))
    t_ref[...] = lo


def _threshold(mx):
    return pl.pallas_call(
        _b_body,
        in_specs=[pl.BlockSpec((Q, NB_REAL), lambda: (0, 0))],
        out_specs=pl.BlockSpec((Q, 1), lambda: (0, 0)),
        out_shape=jax.ShapeDtypeStruct((Q, 1), jnp.float32),
    )(mx)


def _sc_body(d2_hbm, mx_hbm, t_hbm, ov_hbm, oi_hbm,
             mx_v, t_v, ids_v, rows_v, val_v, idx_v, sem):
    wid = lax.axis_index("s") * 2 + lax.axis_index("c")
    qbase = wid * QPW
    pltpu.sync_copy(t_hbm, t_v)

    def qbody(qi, _):
        q = qbase + qi
        pltpu.sync_copy(mx_hbm.at[q], mx_v)
        tq = plsc.load_gather(t_v, (jnp.full((16,), q, jnp.int32),))

        for jj in range(CAP // 16):
            off = jj * 16
            ids_v[pl.ds(off, 16)] = (off + lax.iota(jnp.int32, 16)) * Q + q
            val_v[pl.ds(off, 16)] = jnp.full((16,), PAD_VAL, jnp.float32)
            idx_v[pl.ds(off, 16)] = jnp.full((16,), 2 ** 30, jnp.int32)

        def mxbody(j, cur):
            m = mx_v[pl.ds(j * 16, 16)]
            mask = m >= tq
            cnt = plsc.cumsum(jnp.where(mask, 1, 0).astype(jnp.int32))
            pos = cur + cnt - 1
            mask = mask & (pos < CAP)
            rid = (j * 16 + lax.iota(jnp.int32, 16)) * Q + q
            plsc.store_scatter(ids_v, (pos,), rid, mask=mask)
            return cur + plsc.all_reduce_population_count(mask)

        cur = lax.fori_loop(0, NB // 16, mxbody, jnp.zeros((16,), jnp.int32))
        n_sb = jnp.minimum(jnp.max(cur), CAP)

        pltpu.async_copy(d2_hbm.at[ids_v], rows_v, sem).wait()

        def rbody(r, cur):
            rid = plsc.load_gather(ids_v, (jnp.full((16,), r, jnp.int32),))
            kbase = ((rid - q) >> 10) * 128
            for c in range(8):
                v = rows_v[r, pl.ds(c * 16, 16)]
                mask = v >= tq
                cnt = plsc.cumsum(jnp.where(mask, 1, 0).astype(jnp.int32))
                pos = cur + cnt - 1
                mask = mask & (pos < CAP)
                plsc.store_scatter(val_v, (pos,), v, mask=mask)
                kidx = kbase + c * 16 + lax.iota(jnp.int32, 16)
                plsc.store_scatter(idx_v, (pos,), kidx, mask=mask)
                cur = cur + plsc.all_reduce_population_count(mask)
            return cur

        lax.fori_loop(0, n_sb, rbody, jnp.zeros((16,), jnp.int32))
        pltpu.sync_copy(val_v, ov_hbm.at[q])
        pltpu.sync_copy(idx_v, oi_hbm.at[q])
        return 0

    lax.fori_loop(0, QPW, qbody, 0)


def _sc_filter(d2, mx, t):
    mesh = plsc.VectorSubcoreMesh(core_axis_name="c", subcore_axis_name="s")
    f = pl.kernel(
        _sc_body,
        out_type=[
            jax.ShapeDtypeStruct((Q, CAP), jnp.float32),
            jax.ShapeDtypeStruct((Q, CAP), jnp.int32),
        ],
        mesh=mesh,
        compiler_params=pltpu.CompilerParams(needs_layout_passes=False),
        scratch_types=[
            pltpu.VMEM((NB,), jnp.float32),
            pltpu.VMEM((Q,), jnp.float32),
            pltpu.VMEM((CAP,), jnp.int32),
            pltpu.VMEM((CAP, 128), jnp.float32),
            pltpu.VMEM((CAP,), jnp.float32),
            pltpu.VMEM((CAP,), jnp.int32),
            pltpu.SemaphoreType.DMA,
        ],
    )
    return f(d2, mx, t)


def _d_body(v_ref, i_ref, ov_ref, oi_ref):
    v = v_ref[...]
    ci = i_ref[...]
    vs, ix = [], []
    for _ in range(K_NB):
        m = jnp.max(v, axis=1, keepdims=True)
        eq = v == m
        cand = jnp.where(eq, ci, 2 ** 30)
        si = jnp.min(cand, axis=1, keepdims=True)
        vs.append(m)
        ix.append(si)
        v = jnp.where(eq & (ci == si), PAD_VAL, v)
    ov_ref[...] = jnp.concatenate(vs, axis=1)
    oi_ref[...] = jnp.concatenate(ix, axis=1)


def _final_topk(cv, cidx):
    return pl.pallas_call(
        _d_body,
        in_specs=[
            pl.BlockSpec((Q, CAP), lambda: (0, 0)),
            pl.BlockSpec((Q, CAP), lambda: (0, 0)),
        ],
        out_specs=[
            pl.BlockSpec((Q, K_NB), lambda: (0, 0)),
            pl.BlockSpec((Q, K_NB), lambda: (0, 0)),
        ],
        out_shape=[
            jax.ShapeDtypeStruct((Q, K_NB), jnp.float32),
            jax.ShapeDtypeStruct((Q, K_NB), jnp.int32),
        ],
    )(cv, cidx)


def kernel(queries, keys):
    kp = jnp.pad(keys, ((0, N_PAD - N_KEYS), (0, 0)), constant_values=1e4)
    kt = kp.T
    d3, mx3 = _scores(queries, kt)
    mx = mx3.transpose(1, 0, 2).reshape(Q, NB)
    t = _threshold(mx[:, :NB_REAL])
    d2 = d3.reshape(NB * Q, 128)
    cv, cidx = _sc_filter(d2, mx, t.reshape(Q))
    return _final_topk(cv, cidx)


# speedup vs baseline: 1.4864x; 1.0902x over previous
import functools

import jax
import jax.numpy as jnp
from jax import lax
from jax.experimental import pallas as pl
from jax.experimental.pallas import tpu as pltpu
from jax.experimental.pallas import tpu_sc as plsc

K_NB = 64
N_KEYS = 100000
N_PAD = 100352
NB = 784
NB_REAL = 782
Q = 1024
D = 128
W = 1024
SB_PER_W = W // 128
CAP = 128
NW = 32
QPW = Q // NW
PAD_VAL = -1e30


def _a_body(q_ref, kt_ref, d_ref, mx_ref):
    q = q_ref[...]
    qsq = jnp.sum(q * q, axis=1, keepdims=True)
    ms = []
    for c in range(SB_PER_W):
        kt = kt_ref[:, c * 128:(c + 1) * 128]
        ksq = jnp.sum(kt * kt, axis=0, keepdims=True)
        dots = jnp.dot(q, kt, preferred_element_type=jnp.float32)
        s = -(qsq + ksq - 2.0 * dots)
        d_ref[c] = s
        ms.append(jnp.max(s, axis=1, keepdims=True))
    mx_ref[0] = jnp.concatenate(ms, axis=1)


def _scores(queries, kt):
    grid = N_PAD // W
    return pl.pallas_call(
        _a_body,
        grid=(grid,),
        in_specs=[
            pl.BlockSpec((Q, D), lambda j: (0, 0)),
            pl.BlockSpec((D, W), lambda j: (0, j)),
        ],
        out_specs=[
            pl.BlockSpec((SB_PER_W, Q, 128), lambda j: (j, 0, 0)),
            pl.BlockSpec((1, Q, SB_PER_W), lambda j: (j, 0, 0)),
        ],
        out_shape=[
            jax.ShapeDtypeStruct((NB, Q, 128), jnp.float32),
            jax.ShapeDtypeStruct((N_PAD // W, Q, SB_PER_W), jnp.float32),
        ],
    )(queries, kt)


def _b_body(mx_ref, t_ref):
    mx = mx_ref[...]
    lo = jnp.min(mx, axis=1, keepdims=True)
    hi = jnp.max(mx, axis=1, keepdims=True) + 1.0

    def it(_, lh):
        lo, hi = lh
        mid = 0.5 * (lo + hi)
        cnt = jnp.sum((mx >= mid).astype(jnp.float32), axis=1, keepdims=True)
        pred = cnt >= float(K_NB)
        return (jnp.where(pred, mid, lo), jnp.where(pred, hi, mid))

    lo, hi = lax.fori_loop(0, 26, it, (lo, hi))
    t_ref[...] = lo


def _threshold(mx):
    return pl.pallas_call(
        _b_body,
        in_specs=[pl.BlockSpec((Q, NB_REAL), lambda: (0, 0))],
        out_specs=pl.BlockSpec((Q, 1), lambda: (0, 0)),
        out_shape=jax.ShapeDtypeStruct((Q, 1), jnp.float32),
    )(mx)


def _sc_body(d2_hbm, mx_hbm, t_hbm, ov_hbm, oi_hbm,
             mx2_v, t_v, ids_v, nsb_v, rows2_v, val_v, idx_v, sem_g, sem_mx):
    wid = lax.axis_index("s") * 2 + lax.axis_index("c")
    qbase = wid * QPW
    pltpu.sync_copy(t_hbm, t_v)
    i32 = jnp.int32

    def mx_scan(qi, mxp):
        q = qbase + qi
        tq = plsc.load_gather(t_v, (jnp.full((16,), q, i32),))
        for jj in range(CAP // 16):
            off = jj * 16
            ids_v[qi, pl.ds(off, 16)] = (off + lax.iota(i32, 16)) * Q + q

        def mxbody(j, cur):
            m = mx2_v[mxp, pl.ds(j * 16, 16)]
            mask = m >= tq
            cnt = plsc.cumsum(jnp.where(mask, 1, 0).astype(i32))
            pos = cur + cnt - 1
            mask = mask & (pos < CAP)
            rid = (j * 16 + lax.iota(i32, 16)) * Q + q
            plsc.store_scatter(ids_v, (jnp.full((16,), qi, i32), pos), rid, mask=mask)
            return cur + plsc.all_reduce_population_count(mask)

        cur = lax.fori_loop(0, NB // 16, mxbody, jnp.zeros((16,), i32))
        plsc.store_scatter(nsb_v, (jnp.full((16,), qi, i32),),
                           jnp.minimum(cur, CAP), mask=lax.iota(i32, 16) == 0)

    def row_scan(qi, p):
        q = qbase + qi
        tq = plsc.load_gather(t_v, (jnp.full((16,), q, i32),))
        n = jnp.max(plsc.load_gather(nsb_v, (jnp.full((16,), qi, i32),)))
        for jj in range(CAP // 16):
            off = jj * 16
            val_v[pl.ds(off, 16)] = jnp.full((16,), PAD_VAL, jnp.float32)
            idx_v[pl.ds(off, 16)] = jnp.full((16,), 2 ** 30, i32)

        def rbody(r, cur):
            rid = plsc.load_gather(
                ids_v, (jnp.full((16,), qi, i32), jnp.full((16,), r, i32)))
            kbase = ((rid - q) >> 10) * 128
            for c in range(8):
                v = rows2_v[p, r, pl.ds(c * 16, 16)]
                mask = v >= tq
                cnt = plsc.cumsum(jnp.where(mask, 1, 0).astype(i32))
                pos = cur + cnt - 1
                mask = mask & (pos < CAP)
                plsc.store_scatter(val_v, (pos,), v, mask=mask)
                kidx = kbase + c * 16 + lax.iota(i32, 16)
                plsc.store_scatter(idx_v, (pos,), kidx, mask=mask)
                cur = cur + plsc.all_reduce_population_count(mask)
            return cur

        lax.fori_loop(0, n, rbody, jnp.zeros((16,), i32))
        pltpu.sync_copy(val_v, ov_hbm.at[q])
        pltpu.sync_copy(idx_v, oi_hbm.at[q])

    pltpu.sync_copy(mx_hbm.at[qbase], mx2_v.at[0])
    mx_scan(0, 0)
    pltpu.async_copy(d2_hbm.at[ids_v.at[0]], rows2_v.at[0], sem_g)
    pltpu.async_copy(mx_hbm.at[qbase + 1], mx2_v.at[1], sem_mx)

    def qbody(qi, _):
        p = lax.rem(qi, 2)
        pm = lax.rem(qi + 1, 2)
        pltpu.make_async_copy(d2_hbm.at[ids_v.at[qi]], rows2_v.at[p], sem_g).wait()

        @pl.when(qi + 1 < QPW)
        def _prep():
            pltpu.make_async_copy(mx_hbm.at[qbase + qi + 1], mx2_v.at[pm], sem_mx).wait()
            mx_scan(qi + 1, pm)
            pltpu.async_copy(d2_hbm.at[ids_v.at[qi + 1]], rows2_v.at[pm], sem_g)

            @pl.when(qi + 2 < QPW)
            def _pref():
                pltpu.async_copy(mx_hbm.at[qbase + qi + 2], mx2_v.at[p], sem_mx)

        row_scan(qi, p)
        return 0

    lax.fori_loop(0, QPW, qbody, 0)


def _sc_filter(d2, mx, t):
    mesh = plsc.VectorSubcoreMesh(core_axis_name="c", subcore_axis_name="s")
    f = pl.kernel(
        _sc_body,
        out_type=[
            jax.ShapeDtypeStruct((Q, CAP), jnp.float32),
            jax.ShapeDtypeStruct((Q, CAP), jnp.int32),
        ],
        mesh=mesh,
        compiler_params=pltpu.CompilerParams(needs_layout_passes=False),
        scratch_types=[
            pltpu.VMEM((2, NB), jnp.float32),
            pltpu.VMEM((Q,), jnp.float32),
            pltpu.VMEM((QPW, CAP), jnp.int32),
            pltpu.VMEM((QPW,), jnp.int32),
            pltpu.VMEM((2, CAP, 128), jnp.float32),
            pltpu.VMEM((CAP,), jnp.float32),
            pltpu.VMEM((CAP,), jnp.int32),
            pltpu.SemaphoreType.DMA,
            pltpu.SemaphoreType.DMA,
        ],
    )
    return f(d2, mx, t)


def _d_body(v_ref, i_ref, ov_ref, oi_ref):
    v = v_ref[...]
    ci = i_ref[...]
    vs, ix = [], []
    for _ in range(K_NB):
        m = jnp.max(v, axis=1, keepdims=True)
        eq = v == m
        cand = jnp.where(eq, ci, 2 ** 30)
        si = jnp.min(cand, axis=1, keepdims=True)
        vs.append(m)
        ix.append(si)
        v = jnp.where(eq & (ci == si), PAD_VAL, v)
    ov_ref[...] = jnp.concatenate(vs, axis=1)
    oi_ref[...] = jnp.concatenate(ix, axis=1)


def _final_topk(cv, cidx):
    return pl.pallas_call(
        _d_body,
        in_specs=[
            pl.BlockSpec((Q, CAP), lambda: (0, 0)),
            pl.BlockSpec((Q, CAP), lambda: (0, 0)),
        ],
        out_specs=[
            pl.BlockSpec((Q, K_NB), lambda: (0, 0)),
            pl.BlockSpec((Q, K_NB), lambda: (0, 0)),
        ],
        out_shape=[
            jax.ShapeDtypeStruct((Q, K_NB), jnp.float32),
            jax.ShapeDtypeStruct((Q, K_NB), jnp.int32),
        ],
    )(cv, cidx)


def kernel(queries, keys):
    kp = jnp.pad(keys, ((0, N_PAD - N_KEYS), (0, 0)), constant_values=1e4)
    kt = kp.T
    d3, mx3 = _scores(queries, kt)
    mx = mx3.transpose(1, 0, 2).reshape(Q, NB)
    t = _threshold(mx[:, :NB_REAL])
    d2 = d3.reshape(NB * Q, 128)
    cv, cidx = _sc_filter(d2, mx, t.reshape(Q))
    return _final_topk(cv, cidx)
